# baseline (speedup 1.0000x reference)
import functools

import jax
import jax.numpy as jnp
from jax import lax
from jax.experimental import pallas as pl
from jax.experimental.pallas import tpu as pltpu


def _compute_t(w_block, *, g: int, c: int, d: int, T: int, eps: float,
               beta: float):
    W = w_block.astype(jnp.float32)
    gi = lax.broadcasted_iota(jnp.int32, (g, c, d), 0)
    ri = lax.broadcasted_iota(jnp.int32, (g, c, d), 1)
    ci = lax.broadcasted_iota(jnp.int32, (g, c, d), 2)
    eye_flat = (ci == gi * c + ri).astype(jnp.float32)
    Z = beta * W + (1.0 - beta) * eye_flat

    mean = jnp.sum(Z, axis=-1, keepdims=True) * (1.0 / d)
    Zc = Z - mean

    S = lax.dot_general(Zc, Zc, (((2,), (2,)), ((0,), (0,))),
                        preferred_element_type=jnp.float32)

    r_i = lax.broadcasted_iota(jnp.int32, (c, c), 0)
    c_i = lax.broadcasted_iota(jnp.int32, (c, c), 1)
    eye = (r_i == c_i).astype(jnp.float32)
    S = S + eps * eye[None, :, :]

    sumsq = jnp.sum(S * S, axis=(1, 2), keepdims=True)
    inv_norm = lax.rsqrt(sumsq)
    S = S * inv_norm

    dn = (((2,), (1,)), ((0,), (0,)))
    B = 1.5 * eye[None, :, :] - 0.5 * S
    for _ in range(T - 1):
        B2 = lax.dot_general(B, B, dn, preferred_element_type=jnp.float32)
        BS = lax.dot_general(B, S, dn, preferred_element_type=jnp.float32)
        B = 1.5 * B - 0.5 * lax.dot_general(B2, BS, dn,
                                            preferred_element_type=jnp.float32)
    B = B * jnp.sqrt(inv_norm)
    return lax.dot_general(B, Zc, dn, preferred_element_type=jnp.float32)


def _fused_kernel(adj_hbm, x_hbm, w_ref, sw_ref, out_ref,
                  abuf, x_ref, t_ref, sup_ref, swb_ref, stats_ref,
                  in_sem, x_sem,
                  *, tm: int, n_tiles: int, g: int, c: int, d: int, T: int,
                  ortho_eps: float, beta: float, bn_eps: float):
    n = x_ref.shape[0]

    def start_fetch(slot, step):
        pltpu.make_async_copy(adj_hbm.at[pl.ds(step * tm, tm), :],
                              abuf.at[slot], in_sem.at[slot]).start()

    def wait_fetch(slot):
        pltpu.make_async_copy(adj_hbm.at[pl.ds(0, tm), :],
                              abuf.at[slot], in_sem.at[slot]).wait()

    x_copy = pltpu.make_async_copy(x_hbm, x_ref, x_sem)
    x_copy.start()
    for s in range(min(3, n_tiles)):
        start_fetch(s, s)

    t_full = _compute_t(w_ref[...], g=g, c=c, d=d, T=T, eps=ortho_eps,
                        beta=beta)
    for gg in range(g):
        t_ref[gg * c:(gg + 1) * c, :] = t_full[gg].astype(t_ref.dtype)
    swb_ref[...] = sw_ref[...].astype(jnp.bfloat16)
    stats_ref[...] = jnp.zeros_like(stats_ref)
    x_copy.wait()
    xb = x_ref[...].astype(jnp.bfloat16)
    sup = jnp.dot(xb, t_ref[...], preferred_element_type=jnp.float32)
    sup_ref[...] = sup

    def tile_body(i, _):
        slot = lax.rem(i, 4)
        wait_fetch(slot)

        @pl.when(i + 3 < n_tiles)
        def _():
            start_fetch(lax.rem(i + 3, 4), i + 3)

        acc = jnp.dot(abuf[slot], sup_ref[...],
                      preferred_element_type=jnp.float32)
        x_tile = x_ref[pl.ds(i * tm, tm), :].astype(jnp.bfloat16)
        acc = acc + jnp.dot(x_tile, swb_ref[...],
                            preferred_element_type=jnp.float32)
        out_ref[pl.ds(i * tm, tm), :] = acc
        stats_ref[0:2, :] = stats_ref[0:2, :] + jnp.concatenate(
            [jnp.sum(acc, axis=0, keepdims=True),
             jnp.sum(acc * acc, axis=0, keepdims=True)], axis=0)
        return 0

    lax.fori_loop(0, n_tiles, tile_body, 0)

    mean = stats_ref[0, :] * (1.0 / n)
    var = jnp.maximum(stats_ref[1, :] * (1.0 / n) - mean * mean, 0.0)
    inv = lax.rsqrt(var + bn_eps)
    out_ref[...] = (out_ref[...] - mean[None, :]) * inv[None, :]


def kernel(x, adj, weight, self_weight):
    beta, T, g, ortho_eps, bn_eps = 0.5, 5, 4, 1e-5, 1e-5
    n, f_in = x.shape
    f_out = weight.shape[1]
    c = f_in // g
    d = f_out

    tm = min(512, n)
    n_tiles = n // tm

    out = pl.pallas_call(
        functools.partial(_fused_kernel, tm=tm, n_tiles=n_tiles, g=g, c=c,
                          d=d, T=T, ortho_eps=ortho_eps, beta=beta,
                          bn_eps=bn_eps),
        out_shape=jax.ShapeDtypeStruct((n, f_out), jnp.float32),
        grid=(1,),
        in_specs=[pl.BlockSpec(memory_space=pltpu.MemorySpace.HBM),
                  pl.BlockSpec(memory_space=pltpu.MemorySpace.HBM),
                  pl.BlockSpec((g, c, d), lambda i: (0, 0, 0)),
                  pl.BlockSpec((f_in, f_out), lambda i: (0, 0))],
        out_specs=pl.BlockSpec((n, f_out), lambda i: (0, 0)),
        scratch_shapes=[pltpu.VMEM((4, tm, n), jnp.float32),
                        pltpu.VMEM((n, f_in), jnp.float32),
                        pltpu.VMEM((f_in, f_out), jnp.bfloat16),
                        pltpu.VMEM((n, f_out), jnp.float32),
                        pltpu.VMEM((f_in, f_out), jnp.bfloat16),
                        pltpu.VMEM((8, f_out), jnp.float32),
                        pltpu.SemaphoreType.DMA((4,)),
                        pltpu.SemaphoreType.DMA(())],
        compiler_params=pltpu.CompilerParams(
            dimension_semantics=("arbitrary",),
            vmem_limit_bytes=60 * 1024 * 1024),
    )(adj, x, weight.reshape(g, c, d), self_weight)
    return out


# speedup vs baseline: 1.0218x; 1.0218x over previous
import functools

import jax
import jax.numpy as jnp
from jax import lax
from jax.experimental import pallas as pl
from jax.experimental.pallas import tpu as pltpu


def _compute_t(w_block, *, g: int, c: int, d: int, T: int, eps: float,
               beta: float):
    W = w_block.astype(jnp.float32)
    gi = lax.broadcasted_iota(jnp.int32, (g, c, d), 0)
    ri = lax.broadcasted_iota(jnp.int32, (g, c, d), 1)
    ci = lax.broadcasted_iota(jnp.int32, (g, c, d), 2)
    eye_flat = (ci == gi * c + ri).astype(jnp.float32)
    Z = beta * W + (1.0 - beta) * eye_flat

    mean = jnp.sum(Z, axis=-1, keepdims=True) * (1.0 / d)
    Zc = Z - mean

    S = lax.dot_general(Zc, Zc, (((2,), (2,)), ((0,), (0,))),
                        preferred_element_type=jnp.float32)

    r_i = lax.broadcasted_iota(jnp.int32, (c, c), 0)
    c_i = lax.broadcasted_iota(jnp.int32, (c, c), 1)
    eye = (r_i == c_i).astype(jnp.float32)
    S = S + eps * eye[None, :, :]

    sumsq = jnp.sum(S * S, axis=(1, 2), keepdims=True)
    inv_norm = lax.rsqrt(sumsq)
    S = S * inv_norm

    dn = (((2,), (1,)), ((0,), (0,)))
    B = 1.5 * eye[None, :, :] - 0.5 * S
    for _ in range(T - 1):
        B2 = lax.dot_general(B, B, dn, preferred_element_type=jnp.float32)
        BS = lax.dot_general(B, S, dn, preferred_element_type=jnp.float32)
        B = 1.5 * B - 0.5 * lax.dot_general(B2, BS, dn,
                                            preferred_element_type=jnp.float32)
    B = B * jnp.sqrt(inv_norm)
    return lax.dot_general(B, Zc, dn, preferred_element_type=jnp.float32)


def _fused_kernel(adj_hbm, x_hbm, w_ref, sw_ref, o_hbm,
                  abuf, x_ref, ybuf, t_ref, sup_ref, swb_ref, xsw_ref,
                  stats_ref, in_sem, x_sem, out_sem,
                  *, tm: int, n_tiles: int, g: int, c: int, d: int, T: int,
                  ortho_eps: float, beta: float, bn_eps: float):
    n = x_ref.shape[0]

    def start_fetch(slot, step):
        pltpu.make_async_copy(adj_hbm.at[pl.ds(step * tm, tm), :],
                              abuf.at[slot], in_sem.at[slot]).start()

    def wait_fetch(slot):
        pltpu.make_async_copy(adj_hbm.at[pl.ds(0, tm), :],
                              abuf.at[slot], in_sem.at[slot]).wait()

    x_copy = pltpu.make_async_copy(x_hbm, x_ref, x_sem)
    x_copy.start()
    for s in range(min(3, n_tiles)):
        start_fetch(s, s)

    t_full = _compute_t(w_ref[...], g=g, c=c, d=d, T=T, eps=ortho_eps,
                        beta=beta)
    for gg in range(g):
        t_ref[gg * c:(gg + 1) * c, :] = t_full[gg].astype(t_ref.dtype)
    swb_ref[...] = sw_ref[...].astype(jnp.bfloat16)
    stats_ref[...] = jnp.zeros_like(stats_ref)
    x_copy.wait()
    xb = x_ref[...].astype(jnp.bfloat16)
    sup = jnp.dot(xb, t_ref[...], preferred_element_type=jnp.float32)
    sup_ref[...] = sup.astype(jnp.bfloat16)
    xsw_ref[...] = jnp.dot(xb, swb_ref[...], preferred_element_type=jnp.float32)

    def tile_body(i, _):
        slot = lax.rem(i, 4)
        wait_fetch(slot)

        @pl.when(i + 3 < n_tiles)
        def _():
            start_fetch(lax.rem(i + 3, 4), i + 3)

        adjb = abuf[slot].astype(jnp.bfloat16)
        acc = jnp.dot(adjb, sup_ref[...], preferred_element_type=jnp.float32)
        acc = acc + xsw_ref[pl.ds(i * tm, tm), :]
        ybuf[pl.ds(i * tm, tm), :] = acc
        stats_ref[0:2, :] = stats_ref[0:2, :] + jnp.concatenate(
            [jnp.sum(acc, axis=0, keepdims=True),
             jnp.sum(acc * acc, axis=0, keepdims=True)], axis=0)
        return 0

    lax.fori_loop(0, n_tiles, tile_body, 0)

    mean = stats_ref[0, :] * (1.0 / n)
    var = jnp.maximum(stats_ref[1, :] * (1.0 / n) - mean * mean, 0.0)
    inv = lax.rsqrt(var + bn_eps)
    for k in range(n_tiles):
        sl = pl.ds(k * tm, tm)
        ybuf[sl, :] = (ybuf[sl, :] - mean[None, :]) * inv[None, :]
        pltpu.make_async_copy(ybuf.at[sl, :], o_hbm.at[sl, :],
                              out_sem.at[k]).start()
    for k in range(n_tiles):
        sl = pl.ds(k * tm, tm)
        pltpu.make_async_copy(ybuf.at[sl, :], o_hbm.at[sl, :],
                              out_sem.at[k]).wait()


def kernel(x, adj, weight, self_weight):
    beta, T, g, ortho_eps, bn_eps = 0.5, 5, 4, 1e-5, 1e-5
    n, f_in = x.shape
    f_out = weight.shape[1]
    c = f_in // g
    d = f_out

    tm = min(512, n)
    n_tiles = n // tm

    out = pl.pallas_call(
        functools.partial(_fused_kernel, tm=tm, n_tiles=n_tiles, g=g, c=c,
                          d=d, T=T, ortho_eps=ortho_eps, beta=beta,
                          bn_eps=bn_eps),
        out_shape=jax.ShapeDtypeStruct((n, f_out), jnp.float32),
        grid=(1,),
        in_specs=[pl.BlockSpec(memory_space=pltpu.MemorySpace.HBM),
                  pl.BlockSpec(memory_space=pltpu.MemorySpace.HBM),
                  pl.BlockSpec((g, c, d), lambda i: (0, 0, 0)),
                  pl.BlockSpec((f_in, f_out), lambda i: (0, 0))],
        out_specs=pl.BlockSpec(memory_space=pltpu.MemorySpace.HBM),
        scratch_shapes=[pltpu.VMEM((4, tm, n), jnp.float32),
                        pltpu.VMEM((n, f_in), jnp.float32),
                        pltpu.VMEM((n, f_out), jnp.float32),
                        pltpu.VMEM((f_in, f_out), jnp.bfloat16),
                        pltpu.VMEM((n, f_out), jnp.bfloat16),
                        pltpu.VMEM((f_in, f_out), jnp.bfloat16),
                        pltpu.VMEM((n, f_out), jnp.float32),
                        pltpu.VMEM((8, f_out), jnp.float32),
                        pltpu.SemaphoreType.DMA((4,)),
                        pltpu.SemaphoreType.DMA(()),
                        pltpu.SemaphoreType.DMA((n // min(512, n),))],
        compiler_params=pltpu.CompilerParams(
            dimension_semantics=("arbitrary",),
            vmem_limit_bytes=64 * 1024 * 1024),
    )(adj, x, weight.reshape(g, c, d), self_weight)
    return out
